# Initial kernel scaffold; baseline (speedup 1.0000x reference)
import jax
import jax.numpy as jnp
from jax.experimental import pallas as pl


def kernel(message, message_length, apply_noise):
    raise NotImplementedError("write your pallas kernel here")


# speedup vs baseline: 1.6268x; 1.6268x over previous
import jax
import jax.numpy as jnp
from jax import lax
from jax.experimental import pallas as pl
from jax.experimental.pallas import tpu as pltpu

_P = 0.1


def _delete_mask_const(B, L, dtype=jnp.float32):
    rand = jax.random.uniform(jax.random.key(42), (B, L))
    return (rand < _P).astype(dtype)


def _row_kernel(msg_ref, rand_ref, out_ref):
    m = msg_ref[0]
    L, V = m.shape
    f32 = jnp.float32

    col0 = m[:, 0:1]
    rmax = jnp.max(m, axis=1, keepdims=True)
    nz_col = (rmax > col0).astype(f32)

    i32 = jnp.int32
    iota_col = lax.broadcasted_iota(i32, (L, 1), 0).astype(f32)
    eye = (lax.broadcasted_iota(i32, (L, L), 0) ==
           lax.broadcasted_iota(i32, (L, L), 1)).astype(f32)

    nz_row = lax.dot_general(nz_col, eye, (((0,), (0,)), ((), ())),
                             preferred_element_type=f32)
    randlt = rand_ref[0]
    keep_row = 1.0 - nz_row * randlt

    tri = (lax.broadcasted_iota(i32, (L, L), 0) <=
           lax.broadcasted_iota(i32, (L, L), 1)).astype(f32)
    prefix = jnp.dot(keep_row, tri, preferred_element_type=f32)
    kc = jnp.sum(keep_row)
    dest = prefix - 1.0

    sel = (iota_col == dest).astype(f32) * keep_row
    out = jnp.dot(sel, m, preferred_element_type=f32)

    pad_col = iota_col >= kc
    iota_v = lax.broadcasted_iota(jnp.int32, (1, V), 1)
    out = jnp.where(pad_col & (iota_v == 0), 1.0, out)
    out_ref[0] = out


def kernel(message, message_length, apply_noise):
    del message_length
    B, L, V = message.shape
    randlt = _delete_mask_const(B, L).reshape(B, 1, L)

    out = pl.pallas_call(
        _row_kernel,
        grid=(B,),
        in_specs=[
            pl.BlockSpec((1, L, V), lambda b: (b, 0, 0)),
            pl.BlockSpec((1, 1, L), lambda b: (b, 0, 0)),
        ],
        out_specs=pl.BlockSpec((1, L, V), lambda b: (b, 0, 0)),
        out_shape=jax.ShapeDtypeStruct((B, L, V), jnp.float32),
        compiler_params=pltpu.CompilerParams(
            dimension_semantics=("arbitrary",),
        ),
    )(message, randlt)

    return jnp.where(jnp.asarray(apply_noise) != 0, out, message)
